# baseline (speedup 1.0000x reference)
import jax
import jax.numpy as jnp
from jax.experimental import pallas as pl
from jax.experimental.pallas import tpu as pltpu

N_ROWS = 1024
N_CLASSES = 100000
IGNORE = 0
SMOOTH = 0.1
FILL = SMOOTH / (N_CLASSES - 1)
CONF = 1.0 - SMOOTH

RSPLIT = 2
RB = N_ROWS // RSPLIT
BLK = 2048
GRID = (N_CLASSES + BLK - 1) // BLK


def _wsum(x, col, tt, fill_row):
    w = jnp.where(col == tt, CONF, fill_row)
    xw = x * w
    s = xw[:, 0:128]
    for k in range(1, BLK // 128):
        s = s + xw[:, k * 128:(k + 1) * 128]
    return s


def _loss_body(logit_ref, tgt_ref, out_ref, acc_ref):
    j = pl.program_id(1)

    @pl.when(j == 0)
    def _():
        acc_ref[...] = jnp.zeros_like(acc_ref)

    x = logit_ref[...]
    t = tgt_ref[...]
    ign = t == IGNORE
    tt = jnp.where(ign, -1, t)
    fill_row = jnp.where(ign, 0.0, FILL)
    col = jax.lax.broadcasted_iota(jnp.int32, x.shape, 1) + j * BLK

    @pl.when(j < GRID - 1)
    def _():
        acc_ref[...] += _wsum(x, col, tt, fill_row)

    @pl.when(j == GRID - 1)
    def _():
        xm = jnp.where(col < N_CLASSES, x, 0.0)
        acc_ref[...] += _wsum(xm, col, tt, fill_row)
        out_ref[0, 0, 0] = -jnp.sum(acc_ref[...])


def kernel(logit, target):
    t2 = target.astype(jnp.int32).reshape(N_ROWS, 1)
    res = pl.pallas_call(
        _loss_body,
        grid=(RSPLIT, GRID),
        in_specs=[
            pl.BlockSpec((RB, BLK), lambda i, j: (i, j)),
            pl.BlockSpec((RB, 1), lambda i, j: (i, 0)),
        ],
        out_specs=pl.BlockSpec((1, 1, 1), lambda i, j: (i, 0, 0), memory_space=pltpu.SMEM),
        out_shape=jax.ShapeDtypeStruct((RSPLIT, 1, 1), jnp.float32),
        scratch_shapes=[pltpu.VMEM((RB, 128), jnp.float32)],
        compiler_params=pltpu.CompilerParams(
            dimension_semantics=("parallel", "arbitrary"),
        ),
    )(logit, t2)
    return jnp.sum(res)


# speedup vs baseline: 1.0294x; 1.0294x over previous
import jax
import jax.numpy as jnp
from jax.experimental import pallas as pl
from jax.experimental.pallas import tpu as pltpu

N_ROWS = 1024
N_CLASSES = 100000
IGNORE = 0
SMOOTH = 0.1
FILL = SMOOTH / (N_CLASSES - 1)
CONF = 1.0 - SMOOTH

RB = 16
GRID = N_ROWS // RB


def _loss_body(logit_ref, tgt_ref, out_ref):
    i = pl.program_id(0)

    @pl.when(i == 0)
    def _():
        out_ref[0, 0] = 0.0

    x = logit_ref[...]
    t = tgt_ref[...]
    ign = t == IGNORE
    tt = jnp.where(ign, -1, t)
    fill_row = jnp.where(ign, 0.0, FILL)
    col = jax.lax.broadcasted_iota(jnp.int32, x.shape, 1)
    w = jnp.where(col == tt, CONF, fill_row)
    out_ref[0, 0] += -jnp.sum(x * w)


def kernel(logit, target):
    t2 = target.astype(jnp.int32).reshape(N_ROWS, 1)
    res = pl.pallas_call(
        _loss_body,
        grid=(GRID,),
        in_specs=[
            pl.BlockSpec((RB, N_CLASSES), lambda i: (i, 0)),
            pl.BlockSpec((RB, 1), lambda i: (i, 0)),
        ],
        out_specs=pl.BlockSpec(memory_space=pltpu.SMEM),
        out_shape=jax.ShapeDtypeStruct((1, 1), jnp.float32),
    )(logit, t2)
    return res[0, 0]


# speedup vs baseline: 1.0474x; 1.0175x over previous
import jax
import jax.numpy as jnp
from jax.experimental import pallas as pl
from jax.experimental.pallas import tpu as pltpu

N_ROWS = 1024
N_CLASSES = 100000
IGNORE = 0
SMOOTH = 0.1
FILL = SMOOTH / (N_CLASSES - 1)
CONF = 1.0 - SMOOTH

RB = 16
GRID = N_ROWS // RB
NBUF = 4


def _copy_in(logit_hbm, buf, sems, blk, slot):
    pltpu.make_async_copy(
        logit_hbm.at[pl.ds(blk * RB, RB), :],
        buf.at[slot],
        sems.at[slot],
    ).start()


def _loss_body(logit_hbm, tgt_ref, out_ref, buf, sems):
    i = pl.program_id(0)
    slot = jax.lax.rem(i, NBUF)

    @pl.when(i == 0)
    def _():
        out_ref[0, 0] = 0.0
        for b in range(NBUF):
            _copy_in(logit_hbm, buf, sems, jnp.int32(b), jnp.int32(b))

    pltpu.make_async_copy(
        logit_hbm.at[pl.ds(i * RB, RB), :],
        buf.at[slot],
        sems.at[slot],
    ).wait()

    x = buf[slot]
    t = tgt_ref[...]
    ign = t == IGNORE
    tt = jnp.where(ign, -1, t)
    fill_row = jnp.where(ign, 0.0, FILL)
    col = jax.lax.broadcasted_iota(jnp.int32, x.shape, 1)
    w = jnp.where(col == tt, CONF, fill_row)
    out_ref[0, 0] += -jnp.sum(x * w)

    @pl.when(i + NBUF < GRID)
    def _():
        _copy_in(logit_hbm, buf, sems, i + NBUF, slot)


def kernel(logit, target):
    t2 = target.astype(jnp.int32).reshape(N_ROWS, 1)
    res = pl.pallas_call(
        _loss_body,
        grid=(GRID,),
        in_specs=[
            pl.BlockSpec(memory_space=pltpu.HBM),
            pl.BlockSpec((RB, 1), lambda i: (i, 0)),
        ],
        out_specs=pl.BlockSpec(memory_space=pltpu.SMEM),
        out_shape=jax.ShapeDtypeStruct((1, 1), jnp.float32),
        scratch_shapes=[
            pltpu.VMEM((NBUF, RB, N_CLASSES), jnp.float32),
            pltpu.SemaphoreType.DMA((NBUF,)),
        ],
    )(logit, t2)
    return res[0, 0]


# speedup vs baseline: 1.0886x; 1.0394x over previous
import jax
import jax.numpy as jnp
from jax.experimental import pallas as pl
from jax.experimental.pallas import tpu as pltpu

N_ROWS = 1024
N_CLASSES = 100000
IGNORE = 0
SMOOTH = 0.1
FILL = SMOOTH / (N_CLASSES - 1)
CONF = 1.0 - SMOOTH
DELTA = CONF - FILL

RB = 16
GRID = N_ROWS // RB
NBUF = 4
LANE = 128
NFULL = N_CLASSES // LANE
REM = N_CLASSES - NFULL * LANE
WIN = 512
TSTART = 99584
TW = N_CLASSES - TSTART
SMAXD = TSTART - WIN


def _copy_in(logit_hbm, buf, sems, blk, slot):
    pltpu.make_async_copy(
        logit_hbm.at[pl.ds(blk * RB, RB), :],
        buf.at[slot],
        sems.at[slot],
    ).start()


def _loss_body(tgt_sref, logit_hbm, tgt_ref, out_ref, buf, sems):
    i = pl.program_id(0)
    slot = jax.lax.rem(i, NBUF)

    @pl.when(i == 0)
    def _():
        out_ref[0, 0] = 0.0
        for b in range(NBUF):
            _copy_in(logit_hbm, buf, sems, jnp.int32(b), jnp.int32(b))

    pltpu.make_async_copy(
        logit_hbm.at[pl.ds(i * RB, RB), :],
        buf.at[slot],
        sems.at[slot],
    ).wait()

    x = buf[slot]

    rs_row = jnp.sum(x, axis=1, keepdims=True)
    t = tgt_ref[...]
    fill_row = jnp.where(t == IGNORE, 0.0, FILL)
    fill_part = jnp.sum(fill_row * rs_row)

    lane_iota = jax.lax.broadcasted_iota(jnp.int32, (1, WIN), 1)
    tail_iota = jax.lax.broadcasted_iota(jnp.int32, (1, TW), 1)
    corr = jnp.float32(0.0)
    for r in range(RB):
        t_r = tgt_sref[i * RB + r]
        start = jnp.minimum((t_r // LANE) * LANE, SMAXD)
        xg = buf[slot, pl.ds(r, 1), pl.ds(start, WIN)]
        val = jnp.sum(jnp.where(lane_iota == (t_r - start), xg, 0.0))
        xt = buf[slot, pl.ds(r, 1), TSTART:N_CLASSES]
        val = val + jnp.sum(jnp.where(tail_iota == (t_r - TSTART), xt, 0.0))
        corr = corr + jnp.where(t_r == IGNORE, 0.0, val)

    out_ref[0, 0] += -(fill_part + DELTA * corr)

    @pl.when(i + NBUF < GRID)
    def _():
        _copy_in(logit_hbm, buf, sems, i + NBUF, slot)


def kernel(logit, target):
    t1 = target.astype(jnp.int32)
    res = pl.pallas_call(
        _loss_body,
        grid_spec=pltpu.PrefetchScalarGridSpec(
            num_scalar_prefetch=1,
            grid=(GRID,),
            in_specs=[
                pl.BlockSpec(memory_space=pltpu.HBM),
                pl.BlockSpec((RB, 1), lambda i, t_sref: (i, 0)),
            ],
            out_specs=pl.BlockSpec(memory_space=pltpu.SMEM),
            scratch_shapes=[
                pltpu.VMEM((NBUF, RB, N_CLASSES), jnp.float32),
                pltpu.SemaphoreType.DMA((NBUF,)),
            ],
        ),
        out_shape=jax.ShapeDtypeStruct((1, 1), jnp.float32),
    )(t1, logit, t1.reshape(N_ROWS, 1))
    return res[0, 0]
